# baseline (speedup 1.0000x reference)
import functools
import math

import jax
import jax.numpy as jnp
from jax import lax
from jax.experimental import pallas as pl
from jax.experimental.pallas import tpu as pltpu
from jax.experimental.pallas import tpu_sc as plsc

_NEG_MASK = -10000000000.0
_VERY_NEG = -100000000000.0

_NT = (((1,), (1,)), ((), ()))
_NN = (((1,), (0,)), ((), ()))

_CH = 128
_GRP = 16
_TPAD = 1008


def _sc_gather(head_flat, tail_flat, loc_flat, qw_flat, ent, wordt):
    F = head_flat.shape[0]
    Lc = loc_flat.shape[0]
    Qn = qw_flat.shape[0]
    WD = ent.shape[1]
    info = plsc.get_sparse_core_info()
    nc, nw = info.num_cores, info.num_cores * info.num_subcores

    f_per = F // nw
    ngf = f_per // (_CH * _GRP)
    LCH = 80
    l_per = Lc // nw
    nlc = l_per // LCH
    ngl = nlc // _GRP
    qw_workers = nw // 2
    QCH = Qn // qw_workers

    mesh = plsc.VectorSubcoreMesh(core_axis_name="c", subcore_axis_name="s")

    @functools.partial(
        pl.kernel,
        out_type=(
            jax.ShapeDtypeStruct((F, WD), jnp.bfloat16),
            jax.ShapeDtypeStruct((F, WD), jnp.bfloat16),
            jax.ShapeDtypeStruct((Lc, WD), jnp.bfloat16),
            jax.ShapeDtypeStruct((Qn, WD), jnp.float32),
        ),
        mesh=mesh,
        compiler_params=pltpu.CompilerParams(use_tc_tiling_on_sc=False),
        scratch_types=(
            pltpu.VMEM((f_per,), jnp.int32),
            tuple(pltpu.VMEM((_CH, WD), jnp.bfloat16) for _ in range(_GRP)),
            pltpu.VMEM((QCH, WD), jnp.float32),
            pltpu.SemaphoreType.DMA,
            pltpu.SemaphoreType.DMA,
        ),
    )
    def k(head_h, tail_h, loc_h, qw_h, ent_h, word_h,
          o_head, o_tail, o_loc, o_qw, idx_all, bufs, qbuf, gsem, wsem):
        wid = lax.axis_index("s") * nc + lax.axis_index("c")

        def run_group(tab_h, out_h, base, goff, ch, nch):
            gd, wd = [], []
            for kk in range(nch):
                coff = goff + kk * ch
                dst = bufs[kk] if ch == _CH else bufs[kk].at[pl.ds(0, ch)]
                gd.append(pltpu.async_copy(
                    tab_h.at[idx_all.at[pl.ds(coff, ch)]], dst, gsem))
            for kk in range(nch):
                coff = goff + kk * ch
                src = bufs[kk] if ch == _CH else bufs[kk].at[pl.ds(0, ch)]
                gd[kk].wait()
                wd.append(pltpu.async_copy(
                    src, out_h.at[pl.ds(base + coff, ch)], wsem))
            for d in wd:
                d.wait()

        def gather_table(ids_h, tab_h, out_h, base, total, ch, ngroups, extra):
            pltpu.sync_copy(ids_h.at[pl.ds(base, total)],
                            idx_all.at[pl.ds(0, total)])

            def body(g, carry):
                run_group(tab_h, out_h, base, g * (ch * _GRP), ch, _GRP)
                return carry
            lax.fori_loop(0, ngroups, body, 0)
            if extra:
                run_group(tab_h, out_h, base, ngroups * (ch * _GRP), ch, extra)

        gather_table(head_h, ent_h, o_head, wid * f_per, f_per, _CH, ngf, 0)
        gather_table(tail_h, ent_h, o_tail, wid * f_per, f_per, _CH, ngf, 0)
        gather_table(loc_h, ent_h, o_loc, wid * l_per, l_per, LCH, ngl,
                     nlc - ngl * _GRP)

        @pl.when(wid < qw_workers)
        def _():
            base_q = wid * QCH
            pltpu.sync_copy(qw_h.at[pl.ds(base_q, QCH)],
                            idx_all.at[pl.ds(0, QCH)])
            pltpu.async_copy(
                word_h.at[idx_all.at[pl.ds(0, QCH)]], qbuf, gsem).wait()
            pltpu.sync_copy(qbuf, o_qw.at[pl.ds(base_q, QCH)])

    return k(head_flat, tail_flat, loc_flat, qw_flat, ent, wordt)


def _sc_rel_scores(rel_flat, t_pad):
    N = rel_flat.shape[0]
    Bsz, TW = t_pad.shape
    F = N // Bsz
    info = plsc.get_sparse_core_info()
    nc, nw = info.num_cores, info.num_cores * info.num_subcores
    per = N // nw
    t_flat = t_pad.reshape(-1)
    mesh = plsc.VectorSubcoreMesh(core_axis_name="c", subcore_axis_name="s")

    @functools.partial(
        pl.kernel,
        out_type=jax.ShapeDtypeStruct((N,), jnp.float32),
        mesh=mesh,
        compiler_params=pltpu.CompilerParams(use_tc_tiling_on_sc=False,
                                             needs_layout_passes=False),
        scratch_types=(
            pltpu.VMEM((per,), jnp.int32),
            pltpu.VMEM((TW,), jnp.float32),
            pltpu.VMEM((per,), jnp.float32),
        ),
    )
    def k(ids_h, t_h, o_h, idx_v, t_v, out_v):
        wid = lax.axis_index("s") * nc + lax.axis_index("c")
        base = wid * per
        b = base // F
        pltpu.sync_copy(ids_h.at[pl.ds(base, per)], idx_v)
        pltpu.sync_copy(t_h.at[pl.ds(pl.multiple_of(b * TW, 8), TW)], t_v)

        def body(i, carry):
            ids16 = idx_v[pl.ds(i * 16, 16)]
            out_v[pl.ds(i * 16, 16)] = plsc.load_gather(t_v, [ids16])
            return carry
        lax.fori_loop(0, per // 16, body, 0)
        pltpu.sync_copy(out_v, o_h.at[pl.ds(base, per)])

    return k(rel_flat, t_flat)


def _lstm_prep(xT, wih_t, whh_t, bsum, We, Wr, relemb_pad):
    T, Bsz, WD = xT.shape
    H = We.shape[0]
    TW = relemb_pad.shape[0]

    def body(x_ref, wih_ref, whh_ref, b_ref, we_ref, wr_ref, re_ref,
             q_ref, qe_ref, t_ref):
        def step(t, hc):
            h, c = hc
            xt = x_ref[t]
            gates = (jnp.dot(xt, wih_ref[...], preferred_element_type=jnp.float32)
                     + jnp.dot(h, whh_ref[...], preferred_element_type=jnp.float32)
                     + b_ref[...])
            ig = 1.0 / (1.0 + jnp.exp(-gates[:, :H]))
            fg = 1.0 / (1.0 + jnp.exp(-gates[:, H:2 * H]))
            gg = jnp.tanh(gates[:, 2 * H:3 * H])
            og = 1.0 / (1.0 + jnp.exp(-gates[:, 3 * H:]))
            c2 = fg * c + ig * gg
            h2 = og * jnp.tanh(c2)
            return (h2, c2)

        h0 = jnp.zeros((Bsz, H), jnp.float32)
        h, _ = lax.fori_loop(0, T, step, (h0, h0))
        q_ref[...] = h
        qe_ref[...] = jnp.dot(h, we_ref[...], preferred_element_type=jnp.float32)
        qr = jnp.dot(h, wr_ref[...], preferred_element_type=jnp.float32)
        t_ref[...] = lax.dot_general(qr, re_ref[...], _NT,
                                     preferred_element_type=jnp.float32)

    return pl.pallas_call(
        body,
        out_shape=(
            jax.ShapeDtypeStruct((Bsz, H), jnp.float32),
            jax.ShapeDtypeStruct((Bsz, WD), jnp.float32),
            jax.ShapeDtypeStruct((Bsz, TW), jnp.float32),
        ),
    )(xT, wih_t, whh_t, bsum, We, Wr, relemb_pad)


def _update(q, v, wet, be_row, rwt, rb_row, We, Wr, relemb_pad):
    Bsz, H = q.shape
    WD = We.shape[1]
    TW = relemb_pad.shape[0]

    def body(q_ref, v_ref, wet_ref, be_ref, rwt_ref, rb_ref, we_ref, wr_ref,
             re_ref, qn_ref, qe_ref, t_ref, qb_ref):
        vp = jnp.dot(v_ref[...], wet_ref[...],
                     preferred_element_type=jnp.float32) + be_ref[...]
        qn = jnp.dot(q_ref[...] + vp, rwt_ref[...],
                     preferred_element_type=jnp.float32) + rb_ref[...]
        qn_ref[...] = qn
        qe_ref[...] = jnp.dot(qn, we_ref[...], preferred_element_type=jnp.float32)
        qr = jnp.dot(qn, wr_ref[...], preferred_element_type=jnp.float32)
        t_ref[...] = lax.dot_general(qr, re_ref[...], _NT,
                                     preferred_element_type=jnp.float32)
        qb_ref[...] = jnp.sum(qn * be_ref[...], axis=1, keepdims=True)

    return pl.pallas_call(
        body,
        out_shape=(
            jax.ShapeDtypeStruct((Bsz, H), jnp.float32),
            jax.ShapeDtypeStruct((Bsz, WD), jnp.float32),
            jax.ShapeDtypeStruct((Bsz, TW), jnp.float32),
            jax.ShapeDtypeStruct((Bsz, 1), jnp.float32),
        ),
    )(q, v, wet, be_row, rwt, rb_row, We, Wr, relemb_pad)


def _layer(qe, rs3, head3, tail3, tmask, scale):
    Bsz, F, WD = head3.shape

    def body(qe_ref, rs_ref, h_ref, t_ref, m_ref, v_ref):
        qe_bf = qe_ref[0].astype(jnp.bfloat16)
        s = lax.dot_general(qe_bf, h_ref[0], _NT,
                            preferred_element_type=jnp.float32)
        s = s + rs_ref[0]
        x = (s + (1.0 - m_ref[0]) * _NEG_MASK) * scale
        mx = jnp.max(x, axis=1, keepdims=True)
        e = jnp.exp(x - mx)
        a = (e / jnp.sum(e, axis=1, keepdims=True)).astype(jnp.bfloat16)
        v_ref[0] = lax.dot_general(a, t_ref[0], _NN,
                                   preferred_element_type=jnp.float32)

    return pl.pallas_call(
        body,
        grid=(Bsz,),
        in_specs=[
            pl.BlockSpec((1, 1, WD), lambda b: (b, 0, 0)),
            pl.BlockSpec((1, 1, F), lambda b: (b, 0, 0)),
            pl.BlockSpec((1, F, WD), lambda b: (b, 0, 0)),
            pl.BlockSpec((1, F, WD), lambda b: (b, 0, 0)),
            pl.BlockSpec((1, 1, F), lambda b: (b, 0, 0)),
        ],
        out_specs=pl.BlockSpec((1, 1, WD), lambda b: (b, 0, 0)),
        out_shape=jax.ShapeDtypeStruct((Bsz, 1, WD), jnp.float32),
    )(qe, rs3, head3, tail3, tmask)


def _readout(q1, v1, wet, be_row, rwt1, rb1, We, loc3, loc_ids, n_entity):
    Bsz, L, WD = loc3.shape
    H = q1.shape[2]

    def body(q_ref, v_ref, wet_ref, be_ref, rwt_ref, rb_ref, we_ref,
             rows_ref, ids_ref, o_ref):
        vp = jnp.dot(v_ref[0], wet_ref[...],
                     preferred_element_type=jnp.float32) + be_ref[...]
        q2 = jnp.dot(q_ref[0] + vp, rwt_ref[...],
                     preferred_element_type=jnp.float32) + rb_ref[...]
        qe = jnp.dot(q2, we_ref[...],
                     preferred_element_type=jnp.float32).astype(jnp.bfloat16)
        qb = jnp.sum(q2 * be_ref[...], axis=1, keepdims=True)
        s = lax.dot_general(qe, rows_ref[0], _NT,
                            preferred_element_type=jnp.float32) + qb
        mask = (ids_ref[0] != n_entity).astype(jnp.float32)
        o_ref[0] = s + (1.0 - mask) * _VERY_NEG

    fixed2 = lambda b: (0, 0)
    return pl.pallas_call(
        body,
        grid=(Bsz,),
        in_specs=[
            pl.BlockSpec((1, 1, H), lambda b: (b, 0, 0)),
            pl.BlockSpec((1, 1, WD), lambda b: (b, 0, 0)),
            pl.BlockSpec((WD, H), fixed2),
            pl.BlockSpec((1, H), fixed2),
            pl.BlockSpec((H, H), fixed2),
            pl.BlockSpec((1, H), fixed2),
            pl.BlockSpec((H, WD), fixed2),
            pl.BlockSpec((1, L, WD), lambda b: (b, 0, 0)),
            pl.BlockSpec((1, 1, L), lambda b: (b, 0, 0)),
        ],
        out_specs=pl.BlockSpec((1, 1, L), lambda b: (b, 0, 0)),
        out_shape=jax.ShapeDtypeStruct((Bsz, 1, L), jnp.float32),
    )(q1, v1, wet, be_row, rwt1, rb1, We, loc3, loc_ids)


def kernel(local_entity, head_ids, rel_ids, tail_ids, tripet_mask, query_text, answer_dist,
           entity_emb, relation_emb, word_emb, ent_lin_W, ent_lin_b, rel_lin_W, rel_lin_b,
           R_W, R_b, lstm_Wih, lstm_Whh, lstm_bih, lstm_bhh):
    f32 = jnp.float32
    Bsz, L = local_entity.shape
    F = head_ids.shape[1]
    T = query_text.shape[1]
    H, WD = ent_lin_W.shape
    n_entity = entity_emb.shape[0] - 1
    scale = float(1.0 / math.sqrt(float(H)))

    head_flat = head_ids.reshape(-1).astype(jnp.int32)
    rel_flat = rel_ids.reshape(-1).astype(jnp.int32)
    tail_flat = tail_ids.reshape(-1).astype(jnp.int32)
    loc_flat = local_entity.reshape(-1).astype(jnp.int32)
    qw_flat = query_text.reshape(-1).astype(jnp.int32)

    head_rows, tail_rows, loc_rows, qw_rows = _sc_gather(
        head_flat, tail_flat, loc_flat, qw_flat,
        entity_emb.astype(jnp.bfloat16), word_emb.astype(f32))

    relemb_pad = jnp.pad(relation_emb.astype(f32),
                         ((0, _TPAD - relation_emb.shape[0]), (0, 0)))
    xT = qw_rows.reshape(Bsz, T, WD).transpose(1, 0, 2)
    bsum = (lstm_bih + lstm_bhh).reshape(1, -1).astype(f32)
    wet = ent_lin_W.T.astype(f32)
    be_row = ent_lin_b.reshape(1, -1).astype(f32)
    We = ent_lin_W.astype(f32)
    Wr = rel_lin_W.astype(f32)

    q0, qe0, t0 = _lstm_prep(xT, lstm_Wih.T.astype(f32), lstm_Whh.T.astype(f32),
                             bsum, We, Wr, relemb_pad)

    head3 = head_rows.reshape(Bsz, F, WD)
    tail3 = tail_rows.reshape(Bsz, F, WD)
    tmask = tripet_mask.astype(f32).reshape(Bsz, 1, F)

    rs0 = _sc_rel_scores(rel_flat, t0).reshape(Bsz, 1, F)
    v0 = _layer(qe0.reshape(Bsz, 1, WD), rs0, head3, tail3, tmask, scale)

    q1, qe1, t1, _ = _update(q0, v0.reshape(Bsz, WD), wet, be_row,
                             R_W[0].T.astype(f32), R_b[0].reshape(1, -1).astype(f32),
                             We, Wr, relemb_pad)
    rs1 = _sc_rel_scores(rel_flat, t1).reshape(Bsz, 1, F)
    v1 = _layer(qe1.reshape(Bsz, 1, WD), rs1, head3, tail3, tmask, scale)

    out = _readout(q1.reshape(Bsz, 1, H), v1, wet, be_row,
                   R_W[1].T.astype(f32), R_b[1].reshape(1, -1).astype(f32),
                   We, loc_rows.reshape(Bsz, L, WD),
                   local_entity.astype(jnp.int32).reshape(Bsz, 1, L), n_entity)
    return out.reshape(Bsz, L)


# speedup vs baseline: 1.8995x; 1.8995x over previous
import functools
import math

import jax
import jax.numpy as jnp
from jax import lax
from jax.experimental import pallas as pl
from jax.experimental.pallas import tpu as pltpu
from jax.experimental.pallas import tpu_sc as plsc

_NEG_MASK = -10000000000.0
_VERY_NEG = -100000000000.0

_NT = (((1,), (1,)), ((), ()))
_NN = (((1,), (0,)), ((), ()))

_CH = 128
_GRP = 8
_TPAD = 1008


def _sc_gather(head_flat, tail_flat, loc_flat, qw_flat, ent, wordt):
    F = head_flat.shape[0]
    Lc = loc_flat.shape[0]
    Qn = qw_flat.shape[0]
    WD = ent.shape[1]
    info = plsc.get_sparse_core_info()
    nc, nw = info.num_cores, info.num_cores * info.num_subcores

    f_per = F // nw
    ngf = f_per // (_CH * _GRP)
    LCH = 80
    l_per = Lc // nw
    nlc = l_per // LCH
    ngl = nlc // _GRP
    qw_workers = nw // 2
    QCH = Qn // qw_workers

    mesh = plsc.VectorSubcoreMesh(core_axis_name="c", subcore_axis_name="s")

    @functools.partial(
        pl.kernel,
        out_type=(
            jax.ShapeDtypeStruct((F, WD), jnp.float32),
            jax.ShapeDtypeStruct((F, WD), jnp.float32),
            jax.ShapeDtypeStruct((Lc, WD), jnp.float32),
            jax.ShapeDtypeStruct((Qn, WD), jnp.float32),
        ),
        mesh=mesh,
        compiler_params=pltpu.CompilerParams(use_tc_tiling_on_sc=False),
        scratch_types=(
            pltpu.VMEM((f_per,), jnp.int32),
            tuple(pltpu.VMEM((_CH, WD), jnp.float32) for _ in range(_GRP)),
            pltpu.VMEM((QCH, WD), jnp.float32),
            pltpu.SemaphoreType.DMA,
            pltpu.SemaphoreType.DMA,
        ),
    )
    def k(head_h, tail_h, loc_h, qw_h, ent_h, word_h,
          o_head, o_tail, o_loc, o_qw, idx_all, bufs, qbuf, gsem, wsem):
        wid = lax.axis_index("s") * nc + lax.axis_index("c")

        def run_group(tab_h, out_h, base, goff, ch, nch):
            gd, wd = [], []
            for kk in range(nch):
                coff = goff + kk * ch
                dst = bufs[kk] if ch == _CH else bufs[kk].at[pl.ds(0, ch)]
                gd.append(pltpu.async_copy(
                    tab_h.at[idx_all.at[pl.ds(coff, ch)]], dst, gsem))
            for kk in range(nch):
                coff = goff + kk * ch
                src = bufs[kk] if ch == _CH else bufs[kk].at[pl.ds(0, ch)]
                gd[kk].wait()
                wd.append(pltpu.async_copy(
                    src, out_h.at[pl.ds(base + coff, ch)], wsem))
            for d in wd:
                d.wait()

        def gather_table(ids_h, tab_h, out_h, base, total, ch, ngroups, extra):
            pltpu.sync_copy(ids_h.at[pl.ds(base, total)],
                            idx_all.at[pl.ds(0, total)])

            def body(g, carry):
                run_group(tab_h, out_h, base, g * (ch * _GRP), ch, _GRP)
                return carry
            lax.fori_loop(0, ngroups, body, 0)
            if extra:
                run_group(tab_h, out_h, base, ngroups * (ch * _GRP), ch, extra)

        gather_table(head_h, ent_h, o_head, wid * f_per, f_per, _CH, ngf, 0)
        gather_table(tail_h, ent_h, o_tail, wid * f_per, f_per, _CH, ngf, 0)
        gather_table(loc_h, ent_h, o_loc, wid * l_per, l_per, LCH, ngl,
                     nlc - ngl * _GRP)

        @pl.when(wid < qw_workers)
        def _():
            base_q = wid * QCH
            pltpu.sync_copy(qw_h.at[pl.ds(base_q, QCH)],
                            idx_all.at[pl.ds(0, QCH)])
            pltpu.async_copy(
                word_h.at[idx_all.at[pl.ds(0, QCH)]], qbuf, gsem).wait()
            pltpu.sync_copy(qbuf, o_qw.at[pl.ds(base_q, QCH)])

    return k(head_flat, tail_flat, loc_flat, qw_flat, ent, wordt)


def _sc_rel_scores(rel_flat, t_pad):
    N = rel_flat.shape[0]
    Bsz, TW = t_pad.shape
    F = N // Bsz
    info = plsc.get_sparse_core_info()
    nc, nw = info.num_cores, info.num_cores * info.num_subcores
    per = N // nw
    t_flat = t_pad.reshape(-1)
    mesh = plsc.VectorSubcoreMesh(core_axis_name="c", subcore_axis_name="s")

    @functools.partial(
        pl.kernel,
        out_type=jax.ShapeDtypeStruct((N,), jnp.float32),
        mesh=mesh,
        compiler_params=pltpu.CompilerParams(use_tc_tiling_on_sc=False,
                                             needs_layout_passes=False),
        scratch_types=(
            pltpu.VMEM((per,), jnp.int32),
            pltpu.VMEM((TW,), jnp.float32),
            pltpu.VMEM((per,), jnp.float32),
        ),
    )
    def k(ids_h, t_h, o_h, idx_v, t_v, out_v):
        wid = lax.axis_index("s") * nc + lax.axis_index("c")
        base = wid * per
        b = base // F
        pltpu.sync_copy(ids_h.at[pl.ds(base, per)], idx_v)
        pltpu.sync_copy(t_h.at[pl.ds(pl.multiple_of(b * TW, 8), TW)], t_v)

        def body(i, carry):
            ids16 = idx_v[pl.ds(i * 16, 16)]
            out_v[pl.ds(i * 16, 16)] = plsc.load_gather(t_v, [ids16])
            return carry
        lax.fori_loop(0, per // 16, body, 0)
        pltpu.sync_copy(out_v, o_h.at[pl.ds(base, per)])

    return k(rel_flat, t_flat)


def _lstm_prep(xT, wih_t, whh_t, bsum, We, Wr, relemb_pad):
    T, Bsz, WD = xT.shape
    H = We.shape[0]
    TW = relemb_pad.shape[0]

    def body(x_ref, wih_ref, whh_ref, b_ref, we_ref, wr_ref, re_ref,
             q_ref, qe_ref, t_ref):
        def step(t, hc):
            h, c = hc
            xt = x_ref[t]
            gates = (jnp.dot(xt, wih_ref[...], preferred_element_type=jnp.float32)
                     + jnp.dot(h, whh_ref[...], preferred_element_type=jnp.float32)
                     + b_ref[...])
            ig = 1.0 / (1.0 + jnp.exp(-gates[:, :H]))
            fg = 1.0 / (1.0 + jnp.exp(-gates[:, H:2 * H]))
            gg = jnp.tanh(gates[:, 2 * H:3 * H])
            og = 1.0 / (1.0 + jnp.exp(-gates[:, 3 * H:]))
            c2 = fg * c + ig * gg
            h2 = og * jnp.tanh(c2)
            return (h2, c2)

        h0 = jnp.zeros((Bsz, H), jnp.float32)
        h, _ = lax.fori_loop(0, T, step, (h0, h0))
        q_ref[...] = h
        qe_ref[...] = jnp.dot(h, we_ref[...], preferred_element_type=jnp.float32)
        qr = jnp.dot(h, wr_ref[...], preferred_element_type=jnp.float32)
        t_ref[...] = lax.dot_general(qr, re_ref[...], _NT,
                                     preferred_element_type=jnp.float32)

    return pl.pallas_call(
        body,
        out_shape=(
            jax.ShapeDtypeStruct((Bsz, H), jnp.float32),
            jax.ShapeDtypeStruct((Bsz, WD), jnp.float32),
            jax.ShapeDtypeStruct((Bsz, TW), jnp.float32),
        ),
    )(xT, wih_t, whh_t, bsum, We, Wr, relemb_pad)


def _update(q, v, wet, be_row, rwt, rb_row, We, Wr, relemb_pad):
    Bsz, H = q.shape
    WD = We.shape[1]
    TW = relemb_pad.shape[0]

    def body(q_ref, v_ref, wet_ref, be_ref, rwt_ref, rb_ref, we_ref, wr_ref,
             re_ref, qn_ref, qe_ref, t_ref, qb_ref):
        vp = jnp.dot(v_ref[...], wet_ref[...],
                     preferred_element_type=jnp.float32) + be_ref[...]
        qn = jnp.dot(q_ref[...] + vp, rwt_ref[...],
                     preferred_element_type=jnp.float32) + rb_ref[...]
        qn_ref[...] = qn
        qe_ref[...] = jnp.dot(qn, we_ref[...], preferred_element_type=jnp.float32)
        qr = jnp.dot(qn, wr_ref[...], preferred_element_type=jnp.float32)
        t_ref[...] = lax.dot_general(qr, re_ref[...], _NT,
                                     preferred_element_type=jnp.float32)
        qb_ref[...] = jnp.sum(qn * be_ref[...], axis=1, keepdims=True)

    return pl.pallas_call(
        body,
        out_shape=(
            jax.ShapeDtypeStruct((Bsz, H), jnp.float32),
            jax.ShapeDtypeStruct((Bsz, WD), jnp.float32),
            jax.ShapeDtypeStruct((Bsz, TW), jnp.float32),
            jax.ShapeDtypeStruct((Bsz, 1), jnp.float32),
        ),
    )(q, v, wet, be_row, rwt, rb_row, We, Wr, relemb_pad)


def _layer(qe, rs_e3, rs_o3, head2, tail2, me3, mo3, scale):
    Bsz, F2, WD2 = head2.shape
    WD = WD2 // 2

    def body(qe_ref, rse_ref, rso_ref, h_ref, t_ref, me_ref, mo_ref, v_ref):
        qe1 = qe_ref[0]
        z = jnp.zeros_like(qe1)
        qe_l = jnp.concatenate([qe1, z], axis=1)
        qe_r = jnp.concatenate([z, qe1], axis=1)
        se = lax.dot_general(qe_l, h_ref[0], _NT,
                             preferred_element_type=jnp.float32) + rse_ref[0]
        so = lax.dot_general(qe_r, h_ref[0], _NT,
                             preferred_element_type=jnp.float32) + rso_ref[0]
        xe = (se + (1.0 - me_ref[0]) * _NEG_MASK) * scale
        xo = (so + (1.0 - mo_ref[0]) * _NEG_MASK) * scale
        mx = jnp.maximum(jnp.max(xe, axis=1, keepdims=True),
                         jnp.max(xo, axis=1, keepdims=True))
        ee = jnp.exp(xe - mx)
        eo = jnp.exp(xo - mx)
        z1 = (jnp.sum(ee, axis=1, keepdims=True)
              + jnp.sum(eo, axis=1, keepdims=True))
        ae = ee / z1
        ao = eo / z1
        ve = lax.dot_general(ae, t_ref[0], _NN,
                             preferred_element_type=jnp.float32)
        vo = lax.dot_general(ao, t_ref[0], _NN,
                             preferred_element_type=jnp.float32)
        v_ref[0] = ve[:, :WD] + vo[:, WD:]

    return pl.pallas_call(
        body,
        grid=(Bsz,),
        in_specs=[
            pl.BlockSpec((1, 1, WD), lambda b: (b, 0, 0)),
            pl.BlockSpec((1, 1, F2), lambda b: (b, 0, 0)),
            pl.BlockSpec((1, 1, F2), lambda b: (b, 0, 0)),
            pl.BlockSpec((1, F2, WD2), lambda b: (b, 0, 0)),
            pl.BlockSpec((1, F2, WD2), lambda b: (b, 0, 0)),
            pl.BlockSpec((1, 1, F2), lambda b: (b, 0, 0)),
            pl.BlockSpec((1, 1, F2), lambda b: (b, 0, 0)),
        ],
        out_specs=pl.BlockSpec((1, 1, WD), lambda b: (b, 0, 0)),
        out_shape=jax.ShapeDtypeStruct((Bsz, 1, WD), jnp.float32),
    )(qe, rs_e3, rs_o3, head2, tail2, me3, mo3)


def _readout(q1, v1, wet, be_row, rwt1, rb1, We, loc2, loc_ids, n_entity):
    Bsz, L2, WD2 = loc2.shape
    L = 2 * L2
    WD = WD2 // 2
    H = q1.shape[2]

    def body(q_ref, v_ref, wet_ref, be_ref, rwt_ref, rb_ref, we_ref,
             rows_ref, ids_ref, o_ref):
        vp = jnp.dot(v_ref[0], wet_ref[...],
                     preferred_element_type=jnp.float32) + be_ref[...]
        q2 = jnp.dot(q_ref[0] + vp, rwt_ref[...],
                     preferred_element_type=jnp.float32) + rb_ref[...]
        qe = jnp.dot(q2, we_ref[...], preferred_element_type=jnp.float32)
        qb = jnp.sum(q2 * be_ref[...], axis=1, keepdims=True)
        z = jnp.zeros_like(qe)
        qe_l = jnp.concatenate([qe, z], axis=1)
        qe_r = jnp.concatenate([z, qe], axis=1)
        se = lax.dot_general(qe_l, rows_ref[0], _NT,
                             preferred_element_type=jnp.float32)
        so = lax.dot_general(qe_r, rows_ref[0], _NT,
                             preferred_element_type=jnp.float32)
        s = jnp.concatenate([se, so], axis=1) + qb
        mask = (ids_ref[0] != n_entity).astype(jnp.float32)
        o_ref[0] = s + (1.0 - mask) * _VERY_NEG

    fixed2 = lambda b: (0, 0)
    return pl.pallas_call(
        body,
        grid=(Bsz,),
        in_specs=[
            pl.BlockSpec((1, 1, H), lambda b: (b, 0, 0)),
            pl.BlockSpec((1, 1, WD), lambda b: (b, 0, 0)),
            pl.BlockSpec((WD, H), fixed2),
            pl.BlockSpec((1, H), fixed2),
            pl.BlockSpec((H, H), fixed2),
            pl.BlockSpec((1, H), fixed2),
            pl.BlockSpec((H, WD), fixed2),
            pl.BlockSpec((1, L2, WD2), lambda b: (b, 0, 0)),
            pl.BlockSpec((1, 1, L), lambda b: (b, 0, 0)),
        ],
        out_specs=pl.BlockSpec((1, 1, L), lambda b: (b, 0, 0)),
        out_shape=jax.ShapeDtypeStruct((Bsz, 1, L), jnp.float32),
    )(q1, v1, wet, be_row, rwt1, rb1, We, loc2, loc_ids)


def kernel(local_entity, head_ids, rel_ids, tail_ids, tripet_mask, query_text, answer_dist,
           entity_emb, relation_emb, word_emb, ent_lin_W, ent_lin_b, rel_lin_W, rel_lin_b,
           R_W, R_b, lstm_Wih, lstm_Whh, lstm_bih, lstm_bhh):
    f32 = jnp.float32
    Bsz, L = local_entity.shape
    F = head_ids.shape[1]
    T = query_text.shape[1]
    H, WD = ent_lin_W.shape
    n_entity = entity_emb.shape[0] - 1
    scale = float(1.0 / math.sqrt(float(H)))

    head_flat = head_ids.reshape(-1).astype(jnp.int32)
    rel_flat = rel_ids.reshape(-1).astype(jnp.int32)
    tail_flat = tail_ids.reshape(-1).astype(jnp.int32)
    loc_flat = local_entity.reshape(-1).astype(jnp.int32)
    qw_flat = query_text.reshape(-1).astype(jnp.int32)

    head_rows, tail_rows, loc_rows, qw_rows = _sc_gather(
        head_flat, tail_flat, loc_flat, qw_flat,
        entity_emb.astype(f32), word_emb.astype(f32))

    relemb_pad = jnp.pad(relation_emb.astype(f32),
                         ((0, _TPAD - relation_emb.shape[0]), (0, 0)))
    xT = qw_rows.reshape(Bsz, T, WD).transpose(1, 0, 2)
    bsum = (lstm_bih + lstm_bhh).reshape(1, -1).astype(f32)
    wet = ent_lin_W.T.astype(f32)
    be_row = ent_lin_b.reshape(1, -1).astype(f32)
    We = ent_lin_W.astype(f32)
    Wr = rel_lin_W.astype(f32)

    q0, qe0, t0 = _lstm_prep(xT, lstm_Wih.T.astype(f32), lstm_Whh.T.astype(f32),
                             bsum, We, Wr, relemb_pad)

    F2 = F // 2
    L2 = L // 2
    head2 = head_rows.reshape(Bsz, F2, 2 * WD)
    tail2 = tail_rows.reshape(Bsz, F2, 2 * WD)
    loc2 = loc_rows.reshape(Bsz, L2, 2 * WD)
    tmask2 = tripet_mask.astype(f32).reshape(Bsz, F2, 2)
    me3 = tmask2[:, :, 0].reshape(Bsz, 1, F2)
    mo3 = tmask2[:, :, 1].reshape(Bsz, 1, F2)
    rel_eo = rel_ids.astype(jnp.int32).reshape(Bsz, F2, 2)
    rel_perm = jnp.concatenate([rel_eo[:, :, 0], rel_eo[:, :, 1]], axis=1)
    rel_perm_flat = rel_perm.reshape(-1)
    ids_eo = local_entity.astype(jnp.int32).reshape(Bsz, L2, 2)
    ids_perm = jnp.concatenate([ids_eo[:, :, 0], ids_eo[:, :, 1]],
                               axis=1).reshape(Bsz, 1, L)

    def split_rs(rs_flat):
        rs = rs_flat.reshape(Bsz, 2, F2)
        return rs[:, 0].reshape(Bsz, 1, F2), rs[:, 1].reshape(Bsz, 1, F2)

    rs0e, rs0o = split_rs(_sc_rel_scores(rel_perm_flat, t0))
    v0 = _layer(qe0.reshape(Bsz, 1, WD), rs0e, rs0o, head2, tail2,
                me3, mo3, scale)

    q1, qe1, t1, _ = _update(q0, v0.reshape(Bsz, WD), wet, be_row,
                             R_W[0].T.astype(f32), R_b[0].reshape(1, -1).astype(f32),
                             We, Wr, relemb_pad)
    rs1e, rs1o = split_rs(_sc_rel_scores(rel_perm_flat, t1))
    v1 = _layer(qe1.reshape(Bsz, 1, WD), rs1e, rs1o, head2, tail2,
                me3, mo3, scale)

    out_perm = _readout(q1.reshape(Bsz, 1, H), v1, wet, be_row,
                        R_W[1].T.astype(f32), R_b[1].reshape(1, -1).astype(f32),
                        We, loc2, ids_perm, n_entity).reshape(Bsz, L)
    out = jnp.stack([out_perm[:, :L2], out_perm[:, L2:]],
                    axis=-1).reshape(Bsz, L)
    return out
